# Initial kernel scaffold; baseline (speedup 1.0000x reference)
import jax
import jax.numpy as jnp
from jax.experimental import pallas as pl


def kernel(queries, keys, values, Wq, bq, Wk, bk, Wv, bv, Wo, bo):
    raise NotImplementedError("write your pallas kernel here")


# speedup vs baseline: 1.5262x; 1.5262x over previous
import math

import numpy as np
import jax
import jax.numpy as jnp
from jax.experimental import pallas as pl
from jax.experimental.pallas import tpu as pltpu

D_MODEL = 1024
NHEAD = 16
DK = 64
LQ = 2048
LK = 2048
FACTOR = 5
U_SAMP = min(FACTOR * int(np.ceil(np.log(LK + 1))), LK)
N_SEL = min(FACTOR * int(np.ceil(np.log(LQ + 1))), LQ)

NEG = -1e30

_index_sample = np.asarray(
    jax.random.randint(jax.random.key(42), (LQ, U_SAMP), 0, LK))
_C_COUNTS = np.zeros((LQ, LK), np.float32)
np.add.at(_C_COUNTS, (np.arange(LQ)[:, None], _index_sample), 1.0)
_MASK_NEG = np.where(_C_COUNTS > 0, 0.0, NEG).astype(np.float32)

_BLK_PROJ = 512
_BLK_M = 256


def _proj_body(x_ref, w_ref, b_ref, o_ref):
    o_ref[0] = (
        jnp.dot(x_ref[0], w_ref[0], preferred_element_type=jnp.float32, precision=jax.lax.Precision.HIGHEST)
        + b_ref[0]
    )


def _mstat_body(q_ref, k_ref, c_ref, mn_ref, m_ref):
    c = c_ref[...]
    mn = mn_ref[...]
    for h in range(NHEAD):
        q = q_ref[h]
        k = k_ref[h].astype(jnp.bfloat16).astype(jnp.float32)
        qk = jax.lax.dot_general(
            q, k, (((1,), (1,)), ((), ())),
            preferred_element_type=jnp.float32,
            precision=jax.lax.Precision.HIGHEST)
        mx = jnp.max(qk + mn, axis=1)
        sm = jnp.sum(qk * c, axis=1)
        m_ref[h, :] = mx - sm * (1.0 / LK)


def _attn_body(q_ref, k_ref, v_ref, m_ref, wot_ref,
               idx_ref, vmean_ref, delta_ref):
    m0 = m_ref[0]
    cols = jax.lax.broadcasted_iota(jnp.int32, (1, LQ), 1)
    rows40 = jax.lax.broadcasted_iota(jnp.int32, (N_SEL, LQ), 0)
    cols40 = jax.lax.broadcasted_iota(jnp.int32, (N_SEL, LQ), 1)

    def body(j, carry):
        m, p = carry
        mx = jnp.max(m)
        col = jnp.min(jnp.where(m == mx, cols, LQ))
        p = p + jnp.where((rows40 == j) & (cols40 == col), 1.0, 0.0)
        return jnp.where(cols == col, NEG, m), p

    _, p = jax.lax.fori_loop(
        0, N_SEL, body, (m0, jnp.zeros((N_SEL, LQ), jnp.float32)))

    idxf = jnp.dot(p, cols.astype(jnp.float32).reshape(LQ, 1),
                   preferred_element_type=jnp.float32, precision=jax.lax.Precision.HIGHEST)
    idx_ref[0, 0, :] = idxf.reshape(1, N_SEL).astype(jnp.int32)[0]

    qsel = jnp.dot(p, q_ref[0], preferred_element_type=jnp.float32, precision=jax.lax.Precision.HIGHEST)
    qselb = qsel.astype(jnp.bfloat16).astype(jnp.float32)
    kb = k_ref[0].astype(jnp.bfloat16).astype(jnp.float32)
    scores = jax.lax.dot_general(
        qselb, kb, (((1,), (1,)), ((), ())),
        preferred_element_type=jnp.float32, precision=jax.lax.Precision.HIGHEST) * (1.0 / math.sqrt(DK))
    smax = jnp.max(scores, axis=1, keepdims=True)
    e = jnp.exp(scores - smax)
    attn = e / jnp.sum(e, axis=1, keepdims=True)
    ctx = jnp.dot(attn, v_ref[0], preferred_element_type=jnp.float32, precision=jax.lax.Precision.HIGHEST)

    vmean = jnp.mean(v_ref[0], axis=0, keepdims=True)
    vmean_ref[0, 0, :] = vmean[0]
    ctxb = ctx.astype(jnp.bfloat16).astype(jnp.float32)
    vmb = vmean.astype(jnp.bfloat16).astype(jnp.float32)
    delta_ref[0] = jnp.dot(ctxb - vmb, wot_ref[...],
                           preferred_element_type=jnp.float32, precision=jax.lax.Precision.HIGHEST)


def _out_body(vm_ref, wot_ref, bo_ref, delta_ref, idx_ref, o_ref):
    vmb = vm_ref[...].astype(jnp.bfloat16).astype(jnp.float32)
    base = (jnp.dot(vmb, wot_ref[...],
                    preferred_element_type=jnp.float32, precision=jax.lax.Precision.HIGHEST)
            + bo_ref[...])
    o_ref[...] = jnp.broadcast_to(base, (LQ, D_MODEL))
    for h in range(NHEAD):
        def body(j, _):
            row = idx_ref[h, 0, j]
            o_ref[pl.ds(row, 1), :] = (
                o_ref[pl.ds(row, 1), :] + delta_ref[h, pl.ds(j, 1), :])
            return 0
        jax.lax.fori_loop(0, N_SEL, body, 0)


def kernel(queries, keys, values, Wq, bq, Wk, bk, Wv, bv, Wo, bo):
    f32 = jnp.float32
    x3 = jnp.stack([queries[0], keys[0], values[0]])
    w3 = jnp.stack([Wq.T, Wk.T, Wv.T])
    b3 = jnp.stack([bq, bk, bv]).reshape(3, 1, D_MODEL)

    qkv = pl.pallas_call(
        _proj_body,
        grid=(3, LQ // _BLK_PROJ),
        in_specs=[
            pl.BlockSpec((1, _BLK_PROJ, D_MODEL), lambda i, j: (i, j, 0)),
            pl.BlockSpec((1, D_MODEL, D_MODEL), lambda i, j: (i, 0, 0)),
            pl.BlockSpec((1, 1, D_MODEL), lambda i, j: (i, 0, 0)),
        ],
        out_specs=pl.BlockSpec((1, _BLK_PROJ, D_MODEL), lambda i, j: (i, j, 0)),
        out_shape=jax.ShapeDtypeStruct((3, LQ, D_MODEL), f32),
    )(x3, w3, b3)

    qkv_h = qkv.reshape(3, LQ, NHEAD, DK).transpose(0, 2, 1, 3)
    qh, kh, vh = qkv_h[0], qkv_h[1], qkv_h[2]

    c_const = jnp.asarray(_C_COUNTS)
    mn_const = jnp.asarray(_MASK_NEG)
    m_stat = pl.pallas_call(
        _mstat_body,
        grid=(LQ // _BLK_M,),
        in_specs=[
            pl.BlockSpec((NHEAD, _BLK_M, DK), lambda i: (0, i, 0)),
            pl.BlockSpec((NHEAD, LK, DK), lambda i: (0, 0, 0)),
            pl.BlockSpec((_BLK_M, LK), lambda i: (i, 0)),
            pl.BlockSpec((_BLK_M, LK), lambda i: (i, 0)),
        ],
        out_specs=pl.BlockSpec((NHEAD, _BLK_M), lambda i: (0, i)),
        out_shape=jax.ShapeDtypeStruct((NHEAD, LQ), f32),
    )(qh, kh, c_const, mn_const)

    m3 = m_stat.reshape(NHEAD, 1, LQ)
    wot = Wo.T
    idx, vmean, delta = pl.pallas_call(
        _attn_body,
        grid=(NHEAD,),
        in_specs=[
            pl.BlockSpec((1, LQ, DK), lambda h: (h, 0, 0)),
            pl.BlockSpec((1, LK, DK), lambda h: (h, 0, 0)),
            pl.BlockSpec((1, LK, DK), lambda h: (h, 0, 0)),
            pl.BlockSpec((1, 1, LQ), lambda h: (h, 0, 0)),
            pl.BlockSpec((DK, D_MODEL), lambda h: (h, 0)),
        ],
        out_specs=[
            pl.BlockSpec((1, 1, N_SEL), lambda h: (h, 0, 0)),
            pl.BlockSpec((1, 1, DK), lambda h: (h, 0, 0)),
            pl.BlockSpec((1, N_SEL, D_MODEL), lambda h: (h, 0, 0)),
        ],
        out_shape=[
            jax.ShapeDtypeStruct((NHEAD, 1, N_SEL), jnp.int32),
            jax.ShapeDtypeStruct((NHEAD, 1, DK), f32),
            jax.ShapeDtypeStruct((NHEAD, N_SEL, D_MODEL), f32),
        ],
    )(qh, kh, vh, m3, wot)

    vm_flat = vmean.reshape(1, D_MODEL)
    out = pl.pallas_call(
        _out_body,
        in_specs=[
            pl.BlockSpec((1, D_MODEL), lambda: (0, 0)),
            pl.BlockSpec((D_MODEL, D_MODEL), lambda: (0, 0)),
            pl.BlockSpec((1, D_MODEL), lambda: (0, 0)),
            pl.BlockSpec((NHEAD, N_SEL, D_MODEL), lambda: (0, 0, 0)),
            pl.BlockSpec(memory_space=pltpu.SMEM),
        ],
        out_specs=pl.BlockSpec((LQ, D_MODEL), lambda: (0, 0)),
        out_shape=jax.ShapeDtypeStruct((LQ, D_MODEL), f32),
    )(vm_flat, wot, bo.reshape(1, D_MODEL), delta, idx)

    return out.reshape(1, LQ, D_MODEL)
